# Initial kernel scaffold; baseline (speedup 1.0000x reference)
import jax
import jax.numpy as jnp
from jax.experimental import pallas as pl


def kernel(x, edge_index, batch, W_rel1, b_rel1, W_root1, W_rel2, b_rel2, W_root2, W_lin1, b_lin1, W_lin2, b_lin2):
    raise NotImplementedError("write your pallas kernel here")


# speedup vs baseline: 10.9350x; 10.9350x over previous
import jax
import jax.numpy as jnp
from jax import lax
from jax.experimental import pallas as pl
from jax.experimental.pallas import tpu as pltpu
from jax.experimental.pallas import tpu_sc as plsc

N = 10000
E = 320000
D = 128
C = 10

NC = 2
NS = 16
NW = NC * NS
NPAD = 10240
RPT = NPAD // NS
CHP = 128
NCHP = 80
EPW = NCHP * CHP
EPAD = NW * EPW

BLK = 1024
GRID = NPAD // BLK


def _sc_segsum_body(with_deg, *refs):
    if with_deg:
        (mat_hbm, src_hbm, dstr_hbm, zrow_hbm, zdeg_hbm, ones_hbm,
         out_hbm, dout_hbm,
         dst_v, sidx0, sidx1, rows0, rows1, acc,
         semi0, semi1, semg0, semg1, sems0, sems1, ones_v, dacc) = refs
    else:
        (mat_hbm, src_hbm, dstr_hbm, zrow_hbm,
         out_hbm,
         dst_v, sidx0, sidx1, rows0, rows1, acc,
         semi0, semi1, semg0, semg1, sems0, sems1) = refs
    cid = lax.axis_index("c")
    sid = lax.axis_index("s")
    wid = cid * NS + sid
    base = wid * EPW

    pltpu.sync_copy(zrow_hbm, acc.at[pl.ds(sid * RPT, RPT)])

    pltpu.sync_copy(dstr_hbm.at[wid], dst_v)

    if with_deg:
        pltpu.sync_copy(zdeg_hbm, dacc.at[pl.ds(sid * RPT, RPT)])
        pltpu.sync_copy(ones_hbm, ones_v)

    plsc.subcore_barrier()

    def idx_src(j):
        return src_hbm.at[pl.ds(base + j * CHP, CHP)]

    pltpu.sync_copy(idx_src(0), sidx0)
    pltpu.async_copy(mat_hbm.at[sidx0], rows0, semg0)
    pltpu.async_copy(idx_src(1), sidx1, semi1)

    def stage(j, sidx_c, semi_c, rows_c, semg_c, sems_c,
              sidx_n, semi_n, rows_n, semg_n, sems_n):
        @pl.when(j + 1 < NCHP)
        def _():
            pltpu.make_async_copy(idx_src(j + 1), sidx_n, semi_n).wait()

        pltpu.make_async_copy(mat_hbm.at[sidx_c], rows_c, semg_c).wait()

        @pl.when(j + 1 < NCHP)
        def _():
            @pl.when(j >= 1)
            def _():
                pltpu.make_async_copy(rows_n, acc.at[dst_v.at[j - 1]],
                                      sems_n).wait()

            pltpu.async_copy(mat_hbm.at[sidx_n], rows_n, semg_n)

        @pl.when(j + 2 < NCHP)
        def _():
            pltpu.async_copy(idx_src(j + 2), sidx_c, semi_c)

        pltpu.async_copy(rows_c, acc.at[dst_v.at[j]], sems_c, add=True)
        if with_deg:
            pltpu.sync_copy(ones_v, dacc.at[dst_v.at[j]], add=True)

    def chunk(j, carry):
        @pl.when(j % 2 == 0)
        def _():
            stage(j, sidx0, semi0, rows0, semg0, sems0,
                  sidx1, semi1, rows1, semg1, sems1)

        @pl.when(j % 2 == 1)
        def _():
            stage(j, sidx1, semi1, rows1, semg1, sems1,
                  sidx0, semi0, rows0, semg0, sems0)

        return carry

    lax.fori_loop(0, NCHP, chunk, 0)

    pltpu.make_async_copy(rows0, acc.at[dst_v.at[NCHP - 2]], sems0).wait()
    pltpu.make_async_copy(rows1, acc.at[dst_v.at[NCHP - 1]], sems1).wait()

    plsc.subcore_barrier()

    pltpu.sync_copy(acc.at[pl.ds(sid * RPT, RPT)],
                    out_hbm.at[cid, pl.ds(sid * RPT, RPT)])
    if with_deg:
        pltpu.sync_copy(dacc.at[pl.ds(sid * RPT, RPT)],
                        dout_hbm.at[cid, pl.ds(sid * RPT, RPT)])


def _sc_segsum(mat, srcp, dstr, zrow, zdeg=None, ones=None):
    with_deg = zdeg is not None
    out_type = [jax.ShapeDtypeStruct((NC, NPAD, D), jnp.float32)]
    scratch = [
        pltpu.VMEM((NCHP, CHP), jnp.int32),
        pltpu.VMEM((CHP,), jnp.int32),
        pltpu.VMEM((CHP,), jnp.int32),
        pltpu.VMEM((CHP, D), jnp.float32),
        pltpu.VMEM((CHP, D), jnp.float32),
        pltpu.VMEM_SHARED((NPAD, D), jnp.float32),
        pltpu.SemaphoreType.DMA,
        pltpu.SemaphoreType.DMA,
        pltpu.SemaphoreType.DMA,
        pltpu.SemaphoreType.DMA,
        pltpu.SemaphoreType.DMA,
        pltpu.SemaphoreType.DMA,
    ]
    if with_deg:
        out_type.append(jax.ShapeDtypeStruct((NC, NPAD), jnp.float32))
        scratch += [pltpu.VMEM((CHP,), jnp.float32),
                    pltpu.VMEM_SHARED((NPAD,), jnp.float32)]
    mesh = plsc.VectorSubcoreMesh(core_axis_name="c", subcore_axis_name="s")
    fn = pl.kernel(
        lambda *refs: _sc_segsum_body(with_deg, *refs),
        out_type=tuple(out_type) if with_deg else out_type[0],
        mesh=mesh,
        scratch_types=scratch,
        name="sc_segsum_deg" if with_deg else "sc_segsum",
    )
    if with_deg:
        return fn(mat, srcp, dstr, zrow, zdeg, ones)
    return fn(mat, srcp, dstr, zrow)


def _sc_deg_body(dstr_hbm, zdeg_hbm, ones_hbm, dout_hbm, dst_v, ones_v, dacc):
    cid = lax.axis_index("c")
    sid = lax.axis_index("s")
    wid = cid * NS + sid

    pltpu.sync_copy(zdeg_hbm, dacc.at[pl.ds(sid * RPT, RPT)])
    pltpu.sync_copy(dstr_hbm.at[wid], dst_v)
    pltpu.sync_copy(ones_hbm, ones_v)

    plsc.subcore_barrier()

    def chunk(j, carry):
        pltpu.sync_copy(ones_v, dacc.at[dst_v.at[j]], add=True)
        return carry

    lax.fori_loop(0, NCHP, chunk, 0)

    plsc.subcore_barrier()

    pltpu.sync_copy(dacc.at[pl.ds(sid * RPT, RPT)],
                    dout_hbm.at[cid, pl.ds(sid * RPT, RPT)])


def _sc_deg(dstr, zdeg, ones):
    mesh = plsc.VectorSubcoreMesh(core_axis_name="c", subcore_axis_name="s")
    fn = pl.kernel(
        _sc_deg_body,
        out_type=jax.ShapeDtypeStruct((NC, NPAD), jnp.float32),
        mesh=mesh,
        scratch_types=[
            pltpu.VMEM((NCHP, CHP), jnp.int32),
            pltpu.VMEM((CHP,), jnp.float32),
            pltpu.VMEM_SHARED((NPAD,), jnp.float32),
        ],
        name="sc_deg",
    )
    return fn(dstr, zdeg, ones)


def _mm_t(a, w):
    return lax.dot_general(a, w, (((1,), (1,)), ((), ())),
                           preferred_element_type=jnp.float32)


def _h1_body(p_ref, deg_ref, x_ref, wrel_ref, brel_ref, wroot_ref, out_ref):
    i = pl.program_id(0)
    dd = deg_ref[...]
    deg = (dd[0] + dd[1])[:, None]
    rec = 1.0 / jnp.maximum(deg, 1.0)
    agg = (p_ref[0] + p_ref[1]) * rec
    h = _mm_t(agg, wrel_ref[...]) + brel_ref[...] + _mm_t(x_ref[...], wroot_ref[...])
    h = jnp.maximum(h, 0.0)
    row = i * BLK + lax.broadcasted_iota(jnp.int32, (BLK, 1), 0)
    out_ref[...] = jnp.where(row < N, h, 0.0)


def _tc_h1(P, DEG, xp, W_rel, b_rel, W_root):
    return pl.pallas_call(
        _h1_body,
        grid=(GRID,),
        in_specs=[
            pl.BlockSpec((NC, BLK, D), lambda i: (0, i, 0)),
            pl.BlockSpec((NC, BLK), lambda i: (0, i)),
            pl.BlockSpec((BLK, D), lambda i: (i, 0)),
            pl.BlockSpec((D, D), lambda i: (0, 0)),
            pl.BlockSpec((1, D), lambda i: (0, 0)),
            pl.BlockSpec((D, D), lambda i: (0, 0)),
        ],
        out_specs=pl.BlockSpec((BLK, D), lambda i: (i, 0)),
        out_shape=jax.ShapeDtypeStruct((NPAD, D), jnp.float32),
        name="tc_h1",
    )(P, DEG, xp, W_rel, b_rel, W_root)


def _head_body(p_ref, deg_ref, h1_ref, wrel_ref, brel_ref, wroot_ref,
               wl1_ref, bl1_ref, wl2_ref, bl2_ref, out_ref, acc1, acc2):
    i = pl.program_id(0)

    @pl.when(i == 0)
    def _():
        acc1[...] = jnp.zeros_like(acc1)
        acc2[...] = jnp.zeros_like(acc2)

    h1 = h1_ref[...]
    dd = deg_ref[...]
    deg = (dd[0] + dd[1])[:, None]
    rec = 1.0 / jnp.maximum(deg, 1.0)
    agg = (p_ref[0] + p_ref[1]) * rec
    h2 = _mm_t(agg, wrel_ref[...]) + brel_ref[...] + _mm_t(h1, wroot_ref[...])
    h2 = jnp.maximum(h2, 0.0)
    row = i * BLK + lax.broadcasted_iota(jnp.int32, (BLK, 1), 0)
    h2 = jnp.where(row < N, h2, 0.0)

    acc1[...] += jnp.sum(h1.reshape(BLK // 8, 8, D), axis=0)
    acc2[...] += jnp.sum(h2.reshape(BLK // 8, 8, D), axis=0)

    @pl.when(i == GRID - 1)
    def _():
        s1 = jnp.sum(acc1[...], axis=0, keepdims=True) * (1.0 / N)
        s2 = jnp.sum(acc2[...], axis=0, keepdims=True) * (1.0 / N)
        pooled = jnp.concatenate([s1, s2], axis=1)
        pooled8 = jnp.broadcast_to(pooled, (8, 2 * D))
        z = jnp.maximum(_mm_t(pooled8, wl1_ref[...]) + bl1_ref[...], 0.0)
        logits = _mm_t(z, wl2_ref[...]) + bl2_ref[...]
        m = jnp.max(logits, axis=1, keepdims=True)
        lse = jnp.log(jnp.sum(jnp.exp(logits - m), axis=1, keepdims=True)) + m
        out_ref[...] = (logits - lse)[0:1, :]


def _tc_head(P, DEG, h1, W_rel, b_rel, W_root, W_lin1, b_lin1, W_lin2, b_lin2):
    return pl.pallas_call(
        _head_body,
        grid=(GRID,),
        in_specs=[
            pl.BlockSpec((NC, BLK, D), lambda i: (0, i, 0)),
            pl.BlockSpec((NC, BLK), lambda i: (0, i)),
            pl.BlockSpec((BLK, D), lambda i: (i, 0)),
            pl.BlockSpec((D, D), lambda i: (0, 0)),
            pl.BlockSpec((1, D), lambda i: (0, 0)),
            pl.BlockSpec((D, D), lambda i: (0, 0)),
            pl.BlockSpec((D, 2 * D), lambda i: (0, 0)),
            pl.BlockSpec((1, D), lambda i: (0, 0)),
            pl.BlockSpec((C, D), lambda i: (0, 0)),
            pl.BlockSpec((1, C), lambda i: (0, 0)),
        ],
        out_specs=pl.BlockSpec((1, C), lambda i: (0, 0)),
        out_shape=jax.ShapeDtypeStruct((1, C), jnp.float32),
        scratch_shapes=[pltpu.VMEM((8, D), jnp.float32),
                        pltpu.VMEM((8, D), jnp.float32)],
        name="tc_head",
    )(P, DEG, h1, W_rel, b_rel, W_root, W_lin1, b_lin1, W_lin2, b_lin2)


def kernel(x, edge_index, batch, W_rel1, b_rel1, W_root1,
           W_rel2, b_rel2, W_root2, W_lin1, b_lin1, W_lin2, b_lin2):
    pad_src = N + jnp.arange(EPAD - E, dtype=jnp.int32) % (NPAD - N)
    pad_dst = N + jnp.arange(EPAD - E, dtype=jnp.int32) % (NPAD - N)
    srcp = jnp.concatenate([edge_index[0], pad_src])
    dstr = jnp.concatenate([edge_index[1], pad_dst]).reshape(NW, NCHP, CHP)
    xp = jnp.concatenate([x, jnp.zeros((NPAD - N, D), x.dtype)], axis=0)
    zrow = jnp.zeros((RPT, D), jnp.float32)
    zdeg = jnp.zeros((RPT,), jnp.float32)

    P1, DEG = _sc_segsum(xp, srcp, dstr, zrow, zdeg,
                         jnp.ones((CHP,), jnp.float32))
    h1 = _tc_h1(P1, DEG, xp, W_rel1, b_rel1.reshape(1, D), W_root1)
    P2 = _sc_segsum(h1, srcp, dstr, zrow)
    return _tc_head(P2, DEG, h1, W_rel2, b_rel2.reshape(1, D), W_root2,
                    W_lin1, b_lin1.reshape(1, D), W_lin2, b_lin2.reshape(1, C))


# speedup vs baseline: 10.9996x; 1.0059x over previous
import jax
import jax.numpy as jnp
from jax import lax
from jax.experimental import pallas as pl
from jax.experimental.pallas import tpu as pltpu
from jax.experimental.pallas import tpu_sc as plsc

N = 10000
E = 320000
D = 128
C = 10

NC = 2
NS = 16
NW = NC * NS
NPAD = 10240
RPT = NPAD // NS
CHP = 128
NCHP = 80
EPW = NCHP * CHP
EPAD = NW * EPW

BLK = 1024
GRID = NPAD // BLK


def _sc_segsum_body(with_deg, *refs):
    if with_deg:
        (mat_hbm, src_hbm, dstr_hbm, zrow_hbm, zdeg_hbm, ones_hbm,
         out_hbm, dout_hbm,
         dst_v, sidx0, sidx1, rows0, rows1, acc,
         semi0, semi1, semg0, semg1, sems0, sems1, ones_v, dacc) = refs
    else:
        (mat_hbm, src_hbm, dstr_hbm, zrow_hbm,
         out_hbm,
         dst_v, sidx0, sidx1, rows0, rows1, acc,
         semi0, semi1, semg0, semg1, sems0, sems1) = refs
    cid = lax.axis_index("c")
    sid = lax.axis_index("s")
    wid = cid * NS + sid
    base = wid * EPW

    def idx_src(j):
        return src_hbm.at[pl.ds(base + j * CHP, CHP)]

    pltpu.sync_copy(idx_src(0), sidx0)
    pltpu.async_copy(mat_hbm.at[sidx0], rows0, semg0)
    pltpu.async_copy(idx_src(1), sidx1, semi1)

    pltpu.sync_copy(dstr_hbm.at[wid], dst_v)

    pltpu.sync_copy(zrow_hbm, acc.at[pl.ds(sid * RPT, RPT)])

    if with_deg:
        pltpu.sync_copy(zdeg_hbm, dacc.at[pl.ds(sid * RPT, RPT)])
        pltpu.sync_copy(ones_hbm, ones_v)

    plsc.subcore_barrier()

    def stage(j, sidx_c, semi_c, rows_c, semg_c, sems_c,
              sidx_n, semi_n, rows_n, semg_n, sems_n):
        @pl.when(j + 1 < NCHP)
        def _():
            pltpu.make_async_copy(idx_src(j + 1), sidx_n, semi_n).wait()

        pltpu.make_async_copy(mat_hbm.at[sidx_c], rows_c, semg_c).wait()

        @pl.when(j + 1 < NCHP)
        def _():
            @pl.when(j >= 1)
            def _():
                pltpu.make_async_copy(rows_n, acc.at[dst_v.at[j - 1]],
                                      sems_n).wait()

            pltpu.async_copy(mat_hbm.at[sidx_n], rows_n, semg_n)

        @pl.when(j + 2 < NCHP)
        def _():
            pltpu.async_copy(idx_src(j + 2), sidx_c, semi_c)

        pltpu.async_copy(rows_c, acc.at[dst_v.at[j]], sems_c, add=True)
        if with_deg:
            pltpu.sync_copy(ones_v, dacc.at[dst_v.at[j]], add=True)

    def chunk(j, carry):
        @pl.when(j % 2 == 0)
        def _():
            stage(j, sidx0, semi0, rows0, semg0, sems0,
                  sidx1, semi1, rows1, semg1, sems1)

        @pl.when(j % 2 == 1)
        def _():
            stage(j, sidx1, semi1, rows1, semg1, sems1,
                  sidx0, semi0, rows0, semg0, sems0)

        return carry

    lax.fori_loop(0, NCHP, chunk, 0)

    pltpu.make_async_copy(rows0, acc.at[dst_v.at[NCHP - 2]], sems0).wait()
    pltpu.make_async_copy(rows1, acc.at[dst_v.at[NCHP - 1]], sems1).wait()

    plsc.subcore_barrier()

    pltpu.sync_copy(acc.at[pl.ds(sid * RPT, RPT)],
                    out_hbm.at[cid, pl.ds(sid * RPT, RPT)])
    if with_deg:
        pltpu.sync_copy(dacc.at[pl.ds(sid * RPT, RPT)],
                        dout_hbm.at[cid, pl.ds(sid * RPT, RPT)])


def _sc_segsum(mat, srcp, dstr, zrow, zdeg=None, ones=None):
    with_deg = zdeg is not None
    out_type = [jax.ShapeDtypeStruct((NC, NPAD, D), jnp.float32)]
    scratch = [
        pltpu.VMEM((NCHP, CHP), jnp.int32),
        pltpu.VMEM((CHP,), jnp.int32),
        pltpu.VMEM((CHP,), jnp.int32),
        pltpu.VMEM((CHP, D), jnp.float32),
        pltpu.VMEM((CHP, D), jnp.float32),
        pltpu.VMEM_SHARED((NPAD, D), jnp.float32),
        pltpu.SemaphoreType.DMA,
        pltpu.SemaphoreType.DMA,
        pltpu.SemaphoreType.DMA,
        pltpu.SemaphoreType.DMA,
        pltpu.SemaphoreType.DMA,
        pltpu.SemaphoreType.DMA,
    ]
    if with_deg:
        out_type.append(jax.ShapeDtypeStruct((NC, NPAD), jnp.float32))
        scratch += [pltpu.VMEM((CHP,), jnp.float32),
                    pltpu.VMEM_SHARED((NPAD,), jnp.float32)]
    mesh = plsc.VectorSubcoreMesh(core_axis_name="c", subcore_axis_name="s")
    fn = pl.kernel(
        lambda *refs: _sc_segsum_body(with_deg, *refs),
        out_type=tuple(out_type) if with_deg else out_type[0],
        mesh=mesh,
        scratch_types=scratch,
        name="sc_segsum_deg" if with_deg else "sc_segsum",
    )
    if with_deg:
        return fn(mat, srcp, dstr, zrow, zdeg, ones)
    return fn(mat, srcp, dstr, zrow)


def _sc_deg_body(dstr_hbm, zdeg_hbm, ones_hbm, dout_hbm, dst_v, ones_v, dacc):
    cid = lax.axis_index("c")
    sid = lax.axis_index("s")
    wid = cid * NS + sid

    pltpu.sync_copy(zdeg_hbm, dacc.at[pl.ds(sid * RPT, RPT)])
    pltpu.sync_copy(dstr_hbm.at[wid], dst_v)
    pltpu.sync_copy(ones_hbm, ones_v)

    plsc.subcore_barrier()

    def chunk(j, carry):
        pltpu.sync_copy(ones_v, dacc.at[dst_v.at[j]], add=True)
        return carry

    lax.fori_loop(0, NCHP, chunk, 0)

    plsc.subcore_barrier()

    pltpu.sync_copy(dacc.at[pl.ds(sid * RPT, RPT)],
                    dout_hbm.at[cid, pl.ds(sid * RPT, RPT)])


def _sc_deg(dstr, zdeg, ones):
    mesh = plsc.VectorSubcoreMesh(core_axis_name="c", subcore_axis_name="s")
    fn = pl.kernel(
        _sc_deg_body,
        out_type=jax.ShapeDtypeStruct((NC, NPAD), jnp.float32),
        mesh=mesh,
        scratch_types=[
            pltpu.VMEM((NCHP, CHP), jnp.int32),
            pltpu.VMEM((CHP,), jnp.float32),
            pltpu.VMEM_SHARED((NPAD,), jnp.float32),
        ],
        name="sc_deg",
    )
    return fn(dstr, zdeg, ones)


def _mm_t(a, w):
    return lax.dot_general(a, w, (((1,), (1,)), ((), ())),
                           preferred_element_type=jnp.float32)


def _h1_body(p_ref, deg_ref, x_ref, wrel_ref, brel_ref, wroot_ref, out_ref):
    i = pl.program_id(0)
    dd = deg_ref[...]
    deg = (dd[0] + dd[1])[:, None]
    rec = 1.0 / jnp.maximum(deg, 1.0)
    agg = (p_ref[0] + p_ref[1]) * rec
    h = _mm_t(agg, wrel_ref[...]) + brel_ref[...] + _mm_t(x_ref[...], wroot_ref[...])
    h = jnp.maximum(h, 0.0)
    row = i * BLK + lax.broadcasted_iota(jnp.int32, (BLK, 1), 0)
    out_ref[...] = jnp.where(row < N, h, 0.0)


def _tc_h1(P, DEG, xp, W_rel, b_rel, W_root):
    return pl.pallas_call(
        _h1_body,
        grid=(GRID,),
        in_specs=[
            pl.BlockSpec((NC, BLK, D), lambda i: (0, i, 0)),
            pl.BlockSpec((NC, BLK), lambda i: (0, i)),
            pl.BlockSpec((BLK, D), lambda i: (i, 0)),
            pl.BlockSpec((D, D), lambda i: (0, 0)),
            pl.BlockSpec((1, D), lambda i: (0, 0)),
            pl.BlockSpec((D, D), lambda i: (0, 0)),
        ],
        out_specs=pl.BlockSpec((BLK, D), lambda i: (i, 0)),
        out_shape=jax.ShapeDtypeStruct((NPAD, D), jnp.float32),
        name="tc_h1",
    )(P, DEG, xp, W_rel, b_rel, W_root)


def _head_body(p_ref, deg_ref, h1_ref, wrel_ref, brel_ref, wroot_ref,
               wl1_ref, bl1_ref, wl2_ref, bl2_ref, out_ref, acc1, acc2):
    i = pl.program_id(0)

    @pl.when(i == 0)
    def _():
        acc1[...] = jnp.zeros_like(acc1)
        acc2[...] = jnp.zeros_like(acc2)

    h1 = h1_ref[...]
    dd = deg_ref[...]
    deg = (dd[0] + dd[1])[:, None]
    rec = 1.0 / jnp.maximum(deg, 1.0)
    agg = (p_ref[0] + p_ref[1]) * rec
    h2 = _mm_t(agg, wrel_ref[...]) + brel_ref[...] + _mm_t(h1, wroot_ref[...])
    h2 = jnp.maximum(h2, 0.0)
    row = i * BLK + lax.broadcasted_iota(jnp.int32, (BLK, 1), 0)
    h2 = jnp.where(row < N, h2, 0.0)

    acc1[...] += jnp.sum(h1.reshape(BLK // 8, 8, D), axis=0)
    acc2[...] += jnp.sum(h2.reshape(BLK // 8, 8, D), axis=0)

    @pl.when(i == GRID - 1)
    def _():
        s1 = jnp.sum(acc1[...], axis=0, keepdims=True) * (1.0 / N)
        s2 = jnp.sum(acc2[...], axis=0, keepdims=True) * (1.0 / N)
        pooled = jnp.concatenate([s1, s2], axis=1)
        pooled8 = jnp.broadcast_to(pooled, (8, 2 * D))
        z = jnp.maximum(_mm_t(pooled8, wl1_ref[...]) + bl1_ref[...], 0.0)
        logits = _mm_t(z, wl2_ref[...]) + bl2_ref[...]
        m = jnp.max(logits, axis=1, keepdims=True)
        lse = jnp.log(jnp.sum(jnp.exp(logits - m), axis=1, keepdims=True)) + m
        out_ref[...] = (logits - lse)[0:1, :]


def _tc_head(P, DEG, h1, W_rel, b_rel, W_root, W_lin1, b_lin1, W_lin2, b_lin2):
    return pl.pallas_call(
        _head_body,
        grid=(GRID,),
        in_specs=[
            pl.BlockSpec((NC, BLK, D), lambda i: (0, i, 0)),
            pl.BlockSpec((NC, BLK), lambda i: (0, i)),
            pl.BlockSpec((BLK, D), lambda i: (i, 0)),
            pl.BlockSpec((D, D), lambda i: (0, 0)),
            pl.BlockSpec((1, D), lambda i: (0, 0)),
            pl.BlockSpec((D, D), lambda i: (0, 0)),
            pl.BlockSpec((D, 2 * D), lambda i: (0, 0)),
            pl.BlockSpec((1, D), lambda i: (0, 0)),
            pl.BlockSpec((C, D), lambda i: (0, 0)),
            pl.BlockSpec((1, C), lambda i: (0, 0)),
        ],
        out_specs=pl.BlockSpec((1, C), lambda i: (0, 0)),
        out_shape=jax.ShapeDtypeStruct((1, C), jnp.float32),
        scratch_shapes=[pltpu.VMEM((8, D), jnp.float32),
                        pltpu.VMEM((8, D), jnp.float32)],
        name="tc_head",
    )(P, DEG, h1, W_rel, b_rel, W_root, W_lin1, b_lin1, W_lin2, b_lin2)


def kernel(x, edge_index, batch, W_rel1, b_rel1, W_root1,
           W_rel2, b_rel2, W_root2, W_lin1, b_lin1, W_lin2, b_lin2):
    pad_src = N + jnp.arange(EPAD - E, dtype=jnp.int32) % (NPAD - N)
    pad_dst = N + jnp.arange(EPAD - E, dtype=jnp.int32) % (NPAD - N)
    srcp = jnp.concatenate([edge_index[0], pad_src])
    dstr = jnp.concatenate([edge_index[1], pad_dst]).reshape(NW, NCHP, CHP)
    xp = jnp.concatenate([x, jnp.zeros((NPAD - N, D), x.dtype)], axis=0)
    zrow = jnp.zeros((RPT, D), jnp.float32)
    zdeg = jnp.zeros((RPT,), jnp.float32)

    P1, DEG = _sc_segsum(xp, srcp, dstr, zrow, zdeg,
                         jnp.ones((CHP,), jnp.float32))
    h1 = _tc_h1(P1, DEG, xp, W_rel1, b_rel1.reshape(1, D), W_root1)
    P2 = _sc_segsum(h1, srcp, dstr, zrow)
    return _tc_head(P2, DEG, h1, W_rel2, b_rel2.reshape(1, D), W_root2,
                    W_lin1, b_lin1.reshape(1, D), W_lin2, b_lin2.reshape(1, C))


# speedup vs baseline: 11.0037x; 1.0004x over previous
import jax
import jax.numpy as jnp
from jax import lax
from jax.experimental import pallas as pl
from jax.experimental.pallas import tpu as pltpu
from jax.experimental.pallas import tpu_sc as plsc

N = 10000
E = 320000
D = 128
C = 10

NC = 2
NS = 16
NW = NC * NS
NPAD = 10240
RPT = NPAD // NS
CHP = 128
NCHP = 80
EPW = NCHP * CHP
EPAD = NW * EPW

BLK = 1024
GRID = NPAD // BLK


def _sc_segsum_body(with_deg, *refs):
    if with_deg:
        (mat_hbm, src_hbm, dstr_hbm, zrow_hbm, zdeg_hbm, ones_hbm,
         out_hbm, dout_hbm,
         dst_v, sidx0, sidx1, rows0, rows1, acc,
         semi0, semi1, semg0, semg1, sems0, sems1, ones_v, dacc) = refs
    else:
        (mat_hbm, src_hbm, dstr_hbm, zrow_hbm,
         out_hbm,
         dst_v, sidx0, sidx1, rows0, rows1, acc,
         semi0, semi1, semg0, semg1, sems0, sems1) = refs
    cid = lax.axis_index("c")
    sid = lax.axis_index("s")
    wid = cid * NS + sid
    base = wid * EPW

    def idx_src(j):
        return src_hbm.at[pl.ds(base + j * CHP, CHP)]

    pltpu.sync_copy(idx_src(0), sidx0)
    pltpu.async_copy(mat_hbm.at[sidx0], rows0, semg0)
    pltpu.async_copy(idx_src(1), sidx1, semi1)

    pltpu.sync_copy(dstr_hbm.at[wid], dst_v)

    pltpu.sync_copy(zrow_hbm, acc.at[pl.ds(sid * RPT, RPT)])

    if with_deg:
        pltpu.sync_copy(zdeg_hbm, dacc.at[pl.ds(sid * RPT, RPT)])
        pltpu.sync_copy(ones_hbm, ones_v)

    plsc.subcore_barrier()

    def stage(j, sidx_c, semi_c, rows_c, semg_c, sems_c,
              sidx_n, semi_n, rows_n, semg_n, sems_n):
        @pl.when(j + 1 < NCHP)
        def _():
            pltpu.make_async_copy(idx_src(j + 1), sidx_n, semi_n).wait()

        pltpu.make_async_copy(mat_hbm.at[sidx_c], rows_c, semg_c).wait()

        @pl.when(j + 1 < NCHP)
        def _():
            @pl.when(j >= 1)
            def _():
                pltpu.make_async_copy(rows_n, acc.at[dst_v.at[j - 1]],
                                      sems_n).wait()

            pltpu.async_copy(mat_hbm.at[sidx_n], rows_n, semg_n)

        @pl.when(j + 2 < NCHP)
        def _():
            pltpu.async_copy(idx_src(j + 2), sidx_c, semi_c)

        pltpu.async_copy(rows_c, acc.at[dst_v.at[j]], sems_c, add=True)
        if with_deg:
            pltpu.sync_copy(ones_v, dacc.at[dst_v.at[j]], add=True)

    def chunk(j, carry):
        @pl.when(j % 2 == 0)
        def _():
            stage(j, sidx0, semi0, rows0, semg0, sems0,
                  sidx1, semi1, rows1, semg1, sems1)

        @pl.when(j % 2 == 1)
        def _():
            stage(j, sidx1, semi1, rows1, semg1, sems1,
                  sidx0, semi0, rows0, semg0, sems0)

        return carry

    lax.fori_loop(0, NCHP, chunk, 0)

    pltpu.make_async_copy(rows0, acc.at[dst_v.at[NCHP - 2]], sems0).wait()
    pltpu.make_async_copy(rows1, acc.at[dst_v.at[NCHP - 1]], sems1).wait()

    plsc.subcore_barrier()

    pltpu.sync_copy(acc.at[pl.ds(sid * RPT, RPT)],
                    out_hbm.at[cid, pl.ds(sid * RPT, RPT)])
    if with_deg:
        pltpu.sync_copy(dacc.at[pl.ds(sid * RPT, RPT)],
                        dout_hbm.at[cid, pl.ds(sid * RPT, RPT)])


def _sc_segsum(mat, srcp, dstr, zrow, zdeg=None, ones=None):
    with_deg = zdeg is not None
    out_type = [jax.ShapeDtypeStruct((NC, NPAD, D), jnp.float32)]
    scratch = [
        pltpu.VMEM((NCHP, CHP), jnp.int32),
        pltpu.VMEM((CHP,), jnp.int32),
        pltpu.VMEM((CHP,), jnp.int32),
        pltpu.VMEM((CHP, D), jnp.float32),
        pltpu.VMEM((CHP, D), jnp.float32),
        pltpu.VMEM_SHARED((NPAD, D), jnp.float32),
        pltpu.SemaphoreType.DMA,
        pltpu.SemaphoreType.DMA,
        pltpu.SemaphoreType.DMA,
        pltpu.SemaphoreType.DMA,
        pltpu.SemaphoreType.DMA,
        pltpu.SemaphoreType.DMA,
    ]
    if with_deg:
        out_type.append(jax.ShapeDtypeStruct((NC, NPAD), jnp.float32))
        scratch += [pltpu.VMEM((CHP,), jnp.float32),
                    pltpu.VMEM_SHARED((NPAD,), jnp.float32)]
    mesh = plsc.VectorSubcoreMesh(core_axis_name="c", subcore_axis_name="s")
    fn = pl.kernel(
        lambda *refs: _sc_segsum_body(with_deg, *refs),
        out_type=tuple(out_type) if with_deg else out_type[0],
        mesh=mesh,
        scratch_types=scratch,
        name="sc_segsum_deg" if with_deg else "sc_segsum",
    )
    if with_deg:
        return fn(mat, srcp, dstr, zrow, zdeg, ones)
    return fn(mat, srcp, dstr, zrow)


def _mm_t(a, w):
    return lax.dot_general(a, w, (((1,), (1,)), ((), ())),
                           preferred_element_type=jnp.float32)


def _h1_body(p_ref, deg_ref, x_ref, wrel_ref, brel_ref, wroot_ref, out_ref):
    i = pl.program_id(0)
    dd = deg_ref[...]
    deg = (dd[0] + dd[1])[:, None]
    rec = 1.0 / jnp.maximum(deg, 1.0)
    agg = (p_ref[0] + p_ref[1]) * rec
    h = _mm_t(agg, wrel_ref[...]) + brel_ref[...] + _mm_t(x_ref[...], wroot_ref[...])
    h = jnp.maximum(h, 0.0)
    row = i * BLK + lax.broadcasted_iota(jnp.int32, (BLK, 1), 0)
    out_ref[...] = jnp.where(row < N, h, 0.0)


def _tc_h1(P, DEG, xp, W_rel, b_rel, W_root):
    return pl.pallas_call(
        _h1_body,
        grid=(GRID,),
        in_specs=[
            pl.BlockSpec((NC, BLK, D), lambda i: (0, i, 0)),
            pl.BlockSpec((NC, BLK), lambda i: (0, i)),
            pl.BlockSpec((BLK, D), lambda i: (i, 0)),
            pl.BlockSpec((D, D), lambda i: (0, 0)),
            pl.BlockSpec((1, D), lambda i: (0, 0)),
            pl.BlockSpec((D, D), lambda i: (0, 0)),
        ],
        out_specs=pl.BlockSpec((BLK, D), lambda i: (i, 0)),
        out_shape=jax.ShapeDtypeStruct((NPAD, D), jnp.float32),
        name="tc_h1",
    )(P, DEG, xp, W_rel, b_rel, W_root)


def _head_body(p_ref, deg_ref, h1_ref, wrel_ref, brel_ref, wroot_ref,
               wl1_ref, bl1_ref, wl2_ref, bl2_ref, out_ref, acc1, acc2):
    i = pl.program_id(0)

    @pl.when(i == 0)
    def _():
        acc1[...] = jnp.zeros_like(acc1)
        acc2[...] = jnp.zeros_like(acc2)

    h1 = h1_ref[...]
    dd = deg_ref[...]
    deg = (dd[0] + dd[1])[:, None]
    rec = 1.0 / jnp.maximum(deg, 1.0)
    agg = (p_ref[0] + p_ref[1]) * rec
    h2 = _mm_t(agg, wrel_ref[...]) + brel_ref[...] + _mm_t(h1, wroot_ref[...])
    h2 = jnp.maximum(h2, 0.0)
    row = i * BLK + lax.broadcasted_iota(jnp.int32, (BLK, 1), 0)
    h2 = jnp.where(row < N, h2, 0.0)

    acc1[...] += jnp.sum(h1.reshape(BLK // 8, 8, D), axis=0)
    acc2[...] += jnp.sum(h2.reshape(BLK // 8, 8, D), axis=0)

    @pl.when(i == GRID - 1)
    def _():
        s1 = jnp.sum(acc1[...], axis=0, keepdims=True) * (1.0 / N)
        s2 = jnp.sum(acc2[...], axis=0, keepdims=True) * (1.0 / N)
        pooled = jnp.concatenate([s1, s2], axis=1)
        pooled8 = jnp.broadcast_to(pooled, (8, 2 * D))
        z = jnp.maximum(_mm_t(pooled8, wl1_ref[...]) + bl1_ref[...], 0.0)
        logits = _mm_t(z, wl2_ref[...]) + bl2_ref[...]
        m = jnp.max(logits, axis=1, keepdims=True)
        lse = jnp.log(jnp.sum(jnp.exp(logits - m), axis=1, keepdims=True)) + m
        out_ref[...] = (logits - lse)[0:1, :]


def _tc_head(P, DEG, h1, W_rel, b_rel, W_root, W_lin1, b_lin1, W_lin2, b_lin2):
    return pl.pallas_call(
        _head_body,
        grid=(GRID,),
        in_specs=[
            pl.BlockSpec((NC, BLK, D), lambda i: (0, i, 0)),
            pl.BlockSpec((NC, BLK), lambda i: (0, i)),
            pl.BlockSpec((BLK, D), lambda i: (i, 0)),
            pl.BlockSpec((D, D), lambda i: (0, 0)),
            pl.BlockSpec((1, D), lambda i: (0, 0)),
            pl.BlockSpec((D, D), lambda i: (0, 0)),
            pl.BlockSpec((D, 2 * D), lambda i: (0, 0)),
            pl.BlockSpec((1, D), lambda i: (0, 0)),
            pl.BlockSpec((C, D), lambda i: (0, 0)),
            pl.BlockSpec((1, C), lambda i: (0, 0)),
        ],
        out_specs=pl.BlockSpec((1, C), lambda i: (0, 0)),
        out_shape=jax.ShapeDtypeStruct((1, C), jnp.float32),
        scratch_shapes=[pltpu.VMEM((8, D), jnp.float32),
                        pltpu.VMEM((8, D), jnp.float32)],
        name="tc_head",
    )(P, DEG, h1, W_rel, b_rel, W_root, W_lin1, b_lin1, W_lin2, b_lin2)


def kernel(x, edge_index, batch, W_rel1, b_rel1, W_root1,
           W_rel2, b_rel2, W_root2, W_lin1, b_lin1, W_lin2, b_lin2):
    pad_src = N + jnp.arange(EPAD - E, dtype=jnp.int32) % (NPAD - N)
    pad_dst = N + jnp.arange(EPAD - E, dtype=jnp.int32) % (NPAD - N)
    srcp = jnp.concatenate([edge_index[0], pad_src])
    dstr = jnp.concatenate([edge_index[1], pad_dst]).reshape(NW, NCHP, CHP)
    xp = jnp.concatenate([x, jnp.zeros((NPAD - N, D), x.dtype)], axis=0)
    zrow = jnp.zeros((RPT, D), jnp.float32)
    zdeg = jnp.zeros((RPT,), jnp.float32)

    P1, DEG = _sc_segsum(xp, srcp, dstr, zrow, zdeg,
                         jnp.ones((CHP,), jnp.float32))
    h1 = _tc_h1(P1, DEG, xp, W_rel1, b_rel1.reshape(1, D), W_root1)
    P2 = _sc_segsum(h1, srcp, dstr, zrow)
    return _tc_head(P2, DEG, h1, W_rel2, b_rel2.reshape(1, D), W_root2,
                    W_lin1, b_lin1.reshape(1, D), W_lin2, b_lin2.reshape(1, C))
